# baseline (speedup 1.0000x reference)
import functools

import jax
import jax.numpy as jnp
from jax import lax
from jax.experimental import pallas as pl
from jax.experimental.pallas import tpu as pltpu
from jax.experimental.pallas import tpu_sc as plsc

N_NODES = 10000
D_IN = 128
D_HID = 256
D_OUT = 128

NC = 2
NS = 16
NW = NC * NS
CHUNK = 64
NBUF = 4
ROWS_PER_SUB = 632
N_PAD = NS * ROWS_PER_SUB
IB = 8
L = 16


def _sc_segment_sum(x, col_flat, row_flat, nch0, nch1):
    mesh = plsc.VectorSubcoreMesh(core_axis_name="c", subcore_axis_name="s")
    zsum = jnp.zeros((N_PAD, D_IN), jnp.float32)
    e0 = nch0 * CHUNK
    e1 = nch1 * CHUNK

    @functools.partial(
        pl.kernel,
        out_type=(
            jax.ShapeDtypeStruct((NC, N_PAD, D_IN), jnp.float32),
            jax.ShapeDtypeStruct((NW, 1, N_PAD), jnp.float32),
        ),
        mesh=mesh,
        compiler_params=pltpu.CompilerParams(needs_layout_passes=False),
        scratch_types=(
            [pltpu.VMEM((CHUNK,), jnp.int32) for _ in range(NBUF)]
            + [pltpu.VMEM((CHUNK,), jnp.int32) for _ in range(NBUF)]
            + [pltpu.VMEM((CHUNK, D_IN), jnp.float32) for _ in range(NBUF)]
            + [
                pltpu.VMEM((N_PAD,), jnp.float32),
                pltpu.VMEM_SHARED((N_PAD, D_IN), jnp.float32),
            ]
            + [pltpu.SemaphoreType.DMA for _ in range(NBUF)]
        ),
    )
    def seg(x_hbm, col_hbm, row_hbm, zsum_hbm, psum_hbm, pcnt_hbm, *sc):
        colv = sc[0:NBUF]
        rowv = sc[NBUF:2 * NBUF]
        rows = sc[2 * NBUF:3 * NBUF]
        cntv = sc[3 * NBUF]
        ssum = sc[3 * NBUF + 1]
        sem = sc[3 * NBUF + 2:3 * NBUF + 2 + NBUF]
        c = lax.axis_index("c")
        s = lax.axis_index("s")
        t = c * NS + s

        @pl.when(s == 0)
        def _zero():
            pltpu.sync_copy(zsum_hbm, ssum)

        def zbody(k, carry):
            cntv[pl.ds(k * L, L)] = jnp.zeros((L,), jnp.float32)
            return carry

        lax.fori_loop(0, N_PAD // L, zbody, 0)
        plsc.subcore_barrier()

        ones = jnp.ones((L,), jnp.float32)

        def run_pipeline(base, nch):
            def start(j, b):
                off = pl.multiple_of(base + j * CHUNK, CHUNK)
                pltpu.sync_copy(col_hbm.at[pl.ds(off, CHUNK)], colv[b])
                pltpu.sync_copy(row_hbm.at[pl.ds(off, CHUNK)], rowv[b])
                pltpu.async_copy(x_hbm.at[colv[b]], rows[b], sem[b])

            def drain(b):
                pltpu.make_async_copy(x_hbm.at[pl.ds(0, CHUNK)], rows[b],
                                      sem[b]).wait()
                pltpu.sync_copy(rows[b], ssum.at[rowv[b]], add=True)
                for k in range(CHUNK // L):
                    iv = rowv[b][pl.ds(k * L, L)]
                    plsc.addupdate_scatter(cntv, [iv], ones)

            for b in range(NBUF):
                start(b, b)

            def group(p, carry):
                j0 = p * NBUF
                for b in range(NBUF):
                    drain(b)
                    start(j0 + b + NBUF, b)
                return carry

            lax.fori_loop(0, nch // NBUF - 1, group, 0)
            for b in range(NBUF):
                drain(b)

        @pl.when(c == 0)
        def _run0():
            run_pipeline(s * e0, nch0)

        @pl.when(c == 1)
        def _run1():
            run_pipeline(NS * e0 + s * e1, nch1)

        plsc.subcore_barrier()

        @pl.when(s == 0)
        def _out():
            pltpu.sync_copy(ssum, psum_hbm.at[c])

        pltpu.sync_copy(cntv, pcnt_hbm.at[t, 0])

    return seg(x, col_flat, row_flat, zsum)


CORE0_SHARE = 0.7


def _tc_mlp_body(x_ref, ps_ref, pc_ref, w1a_ref, w1b_ref, b1_ref,
                 w2_ref, b2_ref, o_ref):
    sums = ps_ref[0] + ps_ref[1]
    cnt = jnp.sum(pc_ref[...], axis=1, keepdims=True)
    mean = sums / jnp.maximum(cnt, 1.0)
    h = jnp.dot(x_ref[...], w1a_ref[...], preferred_element_type=jnp.float32)
    h += jnp.dot(mean, w1b_ref[...], preferred_element_type=jnp.float32)
    h = jnp.maximum(h + b1_ref[...], 0.0)
    o_ref[...] = (
        jnp.dot(h, w2_ref[...], preferred_element_type=jnp.float32)
        + b2_ref[...]
    )


def kernel(x, edge_index, W1, b1, W2, b2):
    E = edge_index.shape[1]
    unit = IB * CHUNK
    u_pair = -(-E // (NS * unit))
    u0 = max(2, min(u_pair - 2, round(CORE0_SHARE * u_pair)))
    u0 += u0 % 2
    u1 = u_pair - u0
    nch0, nch1 = u0 * IB, u1 * IB
    pad = NS * u_pair * unit - E

    row = edge_index[0].astype(jnp.int32)
    col = edge_index[1].astype(jnp.int32)
    col_flat = jnp.concatenate([col, jnp.zeros((pad,), jnp.int32)])
    row_flat = jnp.concatenate([row, jnp.full((pad,), N_NODES, jnp.int32)])

    psum, pcnt = _sc_segment_sum(x, col_flat, row_flat, nch0, nch1)

    n = x.shape[0]
    pcnt_t = pcnt.reshape(NW, N_PAD).T[:n, :]

    blk = 400
    grid = n // blk
    out = pl.pallas_call(
        _tc_mlp_body,
        grid=(grid,),
        in_specs=[
            pl.BlockSpec((blk, D_IN), lambda i: (i, 0)),
            pl.BlockSpec((NC, blk, D_IN), lambda i: (0, i, 0)),
            pl.BlockSpec((blk, NW), lambda i: (i, 0)),
            pl.BlockSpec((D_IN, D_HID), lambda i: (0, 0)),
            pl.BlockSpec((D_IN, D_HID), lambda i: (0, 0)),
            pl.BlockSpec((1, D_HID), lambda i: (0, 0)),
            pl.BlockSpec((D_HID, D_OUT), lambda i: (0, 0)),
            pl.BlockSpec((1, D_OUT), lambda i: (0, 0)),
        ],
        out_specs=pl.BlockSpec((blk, D_OUT), lambda i: (i, 0)),
        out_shape=jax.ShapeDtypeStruct((n, D_OUT), jnp.float32),
    )(
        x,
        psum[:, :n, :],
        pcnt_t,
        W1[:D_IN, :],
        W1[D_IN:, :],
        b1.reshape(1, D_HID),
        W2,
        b2.reshape(1, D_OUT),
    )
    return out


# speedup vs baseline: 1.0111x; 1.0111x over previous
import functools

import jax
import jax.numpy as jnp
from jax import lax
from jax.experimental import pallas as pl
from jax.experimental.pallas import tpu as pltpu
from jax.experimental.pallas import tpu_sc as plsc

N_NODES = 10000
D_IN = 128
D_HID = 256
D_OUT = 128

NC = 2
NS = 16
NW = NC * NS
CHUNK = 64
NBUF = 4
ROWS_PER_SUB = 632
N_PAD = NS * ROWS_PER_SUB
IB = 8
L = 16


def _sc_segment_sum(x, col_flat, row_flat, nch0, nch1):
    mesh = plsc.VectorSubcoreMesh(core_axis_name="c", subcore_axis_name="s")
    zsum = jnp.zeros((N_PAD, D_IN), jnp.float32)
    e0 = nch0 * CHUNK
    e1 = nch1 * CHUNK

    @functools.partial(
        pl.kernel,
        out_type=(
            jax.ShapeDtypeStruct((NC, N_PAD, D_IN), jnp.float32),
            jax.ShapeDtypeStruct((NW, 1, N_PAD), jnp.float32),
        ),
        mesh=mesh,
        compiler_params=pltpu.CompilerParams(needs_layout_passes=False),
        scratch_types=(
            [pltpu.VMEM((CHUNK,), jnp.int32) for _ in range(NBUF)]
            + [pltpu.VMEM((CHUNK,), jnp.int32) for _ in range(NBUF)]
            + [pltpu.VMEM((CHUNK, D_IN), jnp.float32) for _ in range(NBUF)]
            + [
                pltpu.VMEM((N_PAD,), jnp.float32),
                pltpu.VMEM_SHARED((N_PAD, D_IN), jnp.float32),
            ]
            + [pltpu.SemaphoreType.DMA for _ in range(NBUF)]
        ),
    )
    def seg(x_hbm, col_hbm, row_hbm, zsum_hbm, psum_hbm, pcnt_hbm, *sc):
        colv = sc[0:NBUF]
        rowv = sc[NBUF:2 * NBUF]
        rows = sc[2 * NBUF:3 * NBUF]
        cntv = sc[3 * NBUF]
        ssum = sc[3 * NBUF + 1]
        sem = sc[3 * NBUF + 2:3 * NBUF + 2 + NBUF]
        c = lax.axis_index("c")
        s = lax.axis_index("s")
        t = c * NS + s

        @pl.when(s == 0)
        def _zero():
            pltpu.sync_copy(zsum_hbm, ssum)

        def zbody(k, carry):
            cntv[pl.ds(k * L, L)] = jnp.zeros((L,), jnp.float32)
            return carry

        lax.fori_loop(0, N_PAD // L, zbody, 0)
        plsc.subcore_barrier()

        ones = jnp.ones((L,), jnp.float32)

        def run_pipeline(base, nch):
            def start(j, b):
                off = pl.multiple_of(base + j * CHUNK, CHUNK)
                pltpu.sync_copy(col_hbm.at[pl.ds(off, CHUNK)], colv[b])
                pltpu.sync_copy(row_hbm.at[pl.ds(off, CHUNK)], rowv[b])
                pltpu.async_copy(x_hbm.at[colv[b]], rows[b], sem[b])

            def drain(b):
                pltpu.make_async_copy(x_hbm.at[pl.ds(0, CHUNK)], rows[b],
                                      sem[b]).wait()
                pltpu.sync_copy(rows[b], ssum.at[rowv[b]], add=True)
                for k in range(CHUNK // L):
                    iv = rowv[b][pl.ds(k * L, L)]
                    plsc.addupdate_scatter(cntv, [iv], ones)

            for b in range(NBUF):
                start(b, b)

            def group(p, carry):
                j0 = p * NBUF
                for b in range(NBUF):
                    drain(b)
                    start(j0 + b + NBUF, b)
                return carry

            lax.fori_loop(0, nch // NBUF - 1, group, 0)
            for b in range(NBUF):
                drain(b)

        @pl.when(c == 0)
        def _run0():
            run_pipeline(s * e0, nch0)

        @pl.when(c == 1)
        def _run1():
            run_pipeline(NS * e0 + s * e1, nch1)

        plsc.subcore_barrier()

        @pl.when(s == 0)
        def _out():
            pltpu.sync_copy(ssum, psum_hbm.at[c])

        pltpu.sync_copy(cntv, pcnt_hbm.at[t, 0])

    return seg(x, col_flat, row_flat, zsum)


CORE0_SHARE = 0.8


def _tc_mlp_body(x_ref, ps_ref, pc_ref, w1a_ref, w1b_ref, b1_ref,
                 w2_ref, b2_ref, o_ref):
    sums = ps_ref[0] + ps_ref[1]
    cnt = jnp.sum(pc_ref[...], axis=1, keepdims=True)
    mean = sums / jnp.maximum(cnt, 1.0)
    h = jnp.dot(x_ref[...], w1a_ref[...], preferred_element_type=jnp.float32)
    h += jnp.dot(mean, w1b_ref[...], preferred_element_type=jnp.float32)
    h = jnp.maximum(h + b1_ref[...], 0.0)
    o_ref[...] = (
        jnp.dot(h, w2_ref[...], preferred_element_type=jnp.float32)
        + b2_ref[...]
    )


def kernel(x, edge_index, W1, b1, W2, b2):
    E = edge_index.shape[1]
    unit = IB * CHUNK
    u_pair = -(-E // (NS * unit))
    u0 = max(2, min(u_pair - 2, round(CORE0_SHARE * u_pair)))
    u0 += u0 % 2
    u1 = u_pair - u0
    nch0, nch1 = u0 * IB, u1 * IB
    pad = NS * u_pair * unit - E

    row = edge_index[0].astype(jnp.int32)
    col = edge_index[1].astype(jnp.int32)
    col_flat = jnp.concatenate([col, jnp.zeros((pad,), jnp.int32)])
    row_flat = jnp.concatenate([row, jnp.full((pad,), N_NODES, jnp.int32)])

    psum, pcnt = _sc_segment_sum(x, col_flat, row_flat, nch0, nch1)

    n = x.shape[0]
    pcnt_t = pcnt.reshape(NW, N_PAD).T[:n, :]

    blk = 400
    grid = n // blk
    out = pl.pallas_call(
        _tc_mlp_body,
        grid=(grid,),
        in_specs=[
            pl.BlockSpec((blk, D_IN), lambda i: (i, 0)),
            pl.BlockSpec((NC, blk, D_IN), lambda i: (0, i, 0)),
            pl.BlockSpec((blk, NW), lambda i: (i, 0)),
            pl.BlockSpec((D_IN, D_HID), lambda i: (0, 0)),
            pl.BlockSpec((D_IN, D_HID), lambda i: (0, 0)),
            pl.BlockSpec((1, D_HID), lambda i: (0, 0)),
            pl.BlockSpec((D_HID, D_OUT), lambda i: (0, 0)),
            pl.BlockSpec((1, D_OUT), lambda i: (0, 0)),
        ],
        out_specs=pl.BlockSpec((blk, D_OUT), lambda i: (i, 0)),
        out_shape=jax.ShapeDtypeStruct((n, D_OUT), jnp.float32),
    )(
        x,
        psum[:, :n, :],
        pcnt_t,
        W1[:D_IN, :],
        W1[D_IN:, :],
        b1.reshape(1, D_HID),
        W2,
        b2.reshape(1, D_OUT),
    )
    return out
